# Initial kernel scaffold; baseline (speedup 1.0000x reference)
import jax
import jax.numpy as jnp
from jax.experimental import pallas as pl


def kernel(x, z_prev, W, S):
    raise NotImplementedError("write your pallas kernel here")


# speedup vs baseline: 16.0349x; 16.0349x over previous
import functools

import jax
import jax.numpy as jnp
from jax.experimental import pallas as pl

_K = 64


def _listalayer_block(x_ref, z_ref, w_ref, s_ref, o_ref):
    upd = jax.lax.dot_general(
        x_ref[...], w_ref[...], (((1,), (1,)), ((), ())),
        preferred_element_type=jnp.float32)
    upd = upd + jax.lax.dot_general(
        z_ref[...], s_ref[...], (((1,), (1,)), ((), ())),
        preferred_element_type=jnp.float32)
    bits = jax.lax.bitcast_convert_type(upd, jnp.int32) & jnp.int32(0x7FFFFFFF)
    rows = upd.shape[0]
    t = jnp.zeros((rows, 1), jnp.int32)
    for b in range(30, -1, -1):
        cand = t | jnp.int32(1 << b)
        cnt = jnp.sum((bits >= cand).astype(jnp.int32), axis=1, keepdims=True)
        t = jnp.where(cnt >= _K, cand, t)
    o_ref[...] = jnp.where(bits >= t, upd, 0.0)


@functools.partial(jax.jit, static_argnames=("block_rows",))
def kernel(x, z_prev, W, S, block_rows: int = 256):
    batch, input_dim = x.shape
    code_dim = W.shape[0]
    grid = (batch // block_rows,)
    return pl.pallas_call(
        _listalayer_block,
        grid=grid,
        in_specs=[
            pl.BlockSpec((block_rows, input_dim), lambda i: (i, 0)),
            pl.BlockSpec((block_rows, code_dim), lambda i: (i, 0)),
            pl.BlockSpec((code_dim, input_dim), lambda i: (0, 0)),
            pl.BlockSpec((code_dim, code_dim), lambda i: (0, 0)),
        ],
        out_specs=pl.BlockSpec((block_rows, code_dim), lambda i: (i, 0)),
        out_shape=jax.ShapeDtypeStruct((batch, code_dim), jnp.float32),
    )(x, z_prev, W, S)
